# baseline (speedup 1.0000x reference)
import functools

import jax
import jax.numpy as jnp
from jax import lax
from jax.experimental import pallas as pl
from jax.experimental.pallas import tpu as pltpu
from jax.experimental.pallas import tpu_sc as plsc

BATCH = 4096
SEQ = 200
HID = 128
EPS = 1e-12

NC = 2
NS = 16
NW = NC * NS
NTOK = BATCH * SEQ
RPW = NTOK // NW
NBLK = RPW // SEQ
L = 16
NJ = HID // L
SPLIT = 104

_mesh = plsc.VectorSubcoreMesh(core_axis_name="c", subcore_axis_name="s")


@functools.partial(
    pl.kernel,
    mesh=_mesh,
    out_type=jax.ShapeDtypeStruct((NTOK, HID), jnp.float32),
    scratch_types=[
        pltpu.VMEM((RPW,), jnp.int32),
        pltpu.VMEM((SEQ, HID), jnp.float32),
        pltpu.VMEM((SEQ, HID), jnp.float32),
        pltpu.VMEM((2, HID), jnp.float32),
        pltpu.VMEM((HID,), jnp.float32),
        pltpu.VMEM((HID,), jnp.float32),
        pltpu.SemaphoreType.DMA,
    ],
)
def _emb_ln_kernel(text_h, table_h, pos_h, type_h, gamma_h, beta_h, out_h,
                   idx_v, add_v, rows_v, type_v, gam_v, bet_v, gsem):
    wid = lax.axis_index("s") * NC + lax.axis_index("c")
    base = wid * RPW

    pltpu.sync_copy(text_h.at[pl.ds(base, RPW)], idx_v)
    pltpu.sync_copy(pos_h.at[pl.ds(0, SEQ)], add_v)
    pltpu.sync_copy(type_h, type_v)
    pltpu.sync_copy(gamma_h, gam_v)
    pltpu.sync_copy(beta_h, bet_v)

    t = [type_v[0, pl.ds(L * j, L)] for j in range(NJ)]

    def add_body(s, carry):
        for j in range(NJ):
            sl = pl.ds(L * j, L)
            add_v[s, sl] = add_v[s, sl] + t[j]
        return carry

    lax.fori_loop(0, SEQ, add_body, 0)

    g = [gam_v[pl.ds(L * j, L)] for j in range(NJ)]
    bt = [bet_v[pl.ds(L * j, L)] for j in range(NJ)]

    lane = lax.iota(jnp.int32, L)
    perms = [(lane ^ k)[:, None] for k in (8, 4, 2, 1)]
    gdn = lax.GatherDimensionNumbers(
        offset_dims=(), collapsed_slice_dims=(0,), start_index_map=(0,))

    def lane_sum(v):
        for p in perms:
            v = v + lax.gather(v, p, gdn, slice_sizes=(1,),
                               mode=lax.GatherScatterMode.PROMISE_IN_BOUNDS)
        return v

    def ln_rows(rows):
        def row_body(i, carry):
            x = [rows[i, pl.ds(L * j, L)] + add_v[i, pl.ds(L * j, L)]
                 for j in range(NJ)]
            s01 = (x[0] + x[1]) + (x[2] + x[3])
            s23 = (x[4] + x[5]) + (x[6] + x[7])
            tot = lane_sum(s01 + s23)
            q = [x[j] * x[j] for j in range(NJ)]
            q01 = (q[0] + q[1]) + (q[2] + q[3])
            q23 = (q[4] + q[5]) + (q[6] + q[7])
            ssq = lane_sum(q01 + q23)
            mu = tot * (1.0 / HID)
            var = ssq * (1.0 / HID) - mu * mu + EPS
            iv = lax.bitcast_convert_type(var, jnp.int32)
            y = lax.bitcast_convert_type(jnp.int32(0x5F3759DF) - (iv >> 1),
                                         jnp.float32)
            y = y * (1.5 - 0.5 * var * y * y)
            y = y * (1.5 - 0.5 * var * y * y)
            y = y * (1.5 - 0.5 * var * y * y)
            for j in range(NJ):
                rows[i, pl.ds(L * j, L)] = (x[j] - mu) * (y * g[j]) + bt[j]
            return carry

        lax.fori_loop(0, SEQ, row_body, 0)

    def blk_body(b, carry):
        rbase = b * SEQ
        h0 = pltpu.async_copy(
            table_h.at[idx_v.at[pl.ds(rbase, SPLIT)]],
            rows_v.at[pl.ds(0, SPLIT)], gsem)
        h1 = pltpu.async_copy(
            table_h.at[idx_v.at[pl.ds(rbase + SPLIT, SEQ - SPLIT)]],
            rows_v.at[pl.ds(SPLIT, SEQ - SPLIT)], gsem)
        h0.wait()
        h1.wait()
        ln_rows(rows_v)
        pltpu.sync_copy(rows_v, out_h.at[pl.ds(base + rbase, SEQ)])
        return carry

    lax.fori_loop(0, NBLK, blk_body, 0)


def kernel(text, text_table, pos_table, type_table, gamma, beta):
    out = _emb_ln_kernel(text.reshape(NTOK), text_table, pos_table,
                         type_table, gamma, beta)
    return out.reshape(BATCH, SEQ, HID)


# speedup vs baseline: 1.1880x; 1.1880x over previous
import functools

import jax
import jax.numpy as jnp
from jax import lax
from jax.experimental import pallas as pl
from jax.experimental.pallas import tpu as pltpu
from jax.experimental.pallas import tpu_sc as plsc

BATCH = 4096
SEQ = 200
HID = 128
EPS = 1e-12

NC = 2
NS = 16
NW = NC * NS
NTOK = BATCH * SEQ
RPW = NTOK // NW
NBLK = RPW // SEQ
L = 16
NJ = HID // L
SPLIT = 104

_mesh = plsc.VectorSubcoreMesh(core_axis_name="c", subcore_axis_name="s")


@functools.partial(
    pl.kernel,
    mesh=_mesh,
    out_type=jax.ShapeDtypeStruct((NTOK, HID), jnp.float32),
    scratch_types=[
        pltpu.VMEM((RPW,), jnp.int32),
        pltpu.VMEM((SEQ, HID), jnp.float32),
        pltpu.VMEM((SEQ, HID), jnp.float32),
        pltpu.VMEM((SEQ, HID), jnp.float32),
        pltpu.VMEM((2, HID), jnp.float32),
        pltpu.VMEM((HID,), jnp.float32),
        pltpu.VMEM((HID,), jnp.float32),
        pltpu.SemaphoreType.DMA,
        pltpu.SemaphoreType.DMA,
        pltpu.SemaphoreType.DMA,
        pltpu.SemaphoreType.DMA,
    ],
)
def _emb_ln_kernel(text_h, table_h, pos_h, type_h, gamma_h, beta_h, out_h,
                   idx_v, add_v, rows0, rows1, type_v, gam_v, bet_v,
                   gsem0, gsem1, osem0, osem1):
    wid = lax.axis_index("s") * NC + lax.axis_index("c")
    base = wid * RPW

    pltpu.sync_copy(text_h.at[pl.ds(base, RPW)], idx_v)
    pltpu.sync_copy(pos_h.at[pl.ds(0, SEQ)], add_v)
    pltpu.sync_copy(type_h, type_v)
    pltpu.sync_copy(gamma_h, gam_v)
    pltpu.sync_copy(beta_h, bet_v)

    t = [type_v[0, pl.ds(L * j, L)] for j in range(NJ)]

    def add_body(s, carry):
        for j in range(NJ):
            sl = pl.ds(L * j, L)
            add_v[s, sl] = add_v[s, sl] + t[j]
        return carry

    lax.fori_loop(0, SEQ, add_body, 0)

    g = [gam_v[pl.ds(L * j, L)] for j in range(NJ)]
    bt = [bet_v[pl.ds(L * j, L)] for j in range(NJ)]

    lane = lax.iota(jnp.int32, L)
    perms = [(lane ^ k)[:, None] for k in (8, 4, 2, 1)]
    gdn = lax.GatherDimensionNumbers(
        offset_dims=(), collapsed_slice_dims=(0,), start_index_map=(0,))

    def lane_sum(v):
        for p in perms:
            v = v + lax.gather(v, p, gdn, slice_sizes=(1,),
                               mode=lax.GatherScatterMode.PROMISE_IN_BOUNDS)
        return v

    def ln_rows(rows):
        def row_body(i, carry):
            x = [rows[i, pl.ds(L * j, L)] + add_v[i, pl.ds(L * j, L)]
                 for j in range(NJ)]
            s01 = (x[0] + x[1]) + (x[2] + x[3])
            s23 = (x[4] + x[5]) + (x[6] + x[7])
            tot = lane_sum(s01 + s23)
            q = [x[j] * x[j] for j in range(NJ)]
            q01 = (q[0] + q[1]) + (q[2] + q[3])
            q23 = (q[4] + q[5]) + (q[6] + q[7])
            ssq = lane_sum(q01 + q23)
            mu = tot * (1.0 / HID)
            var = ssq * (1.0 / HID) - mu * mu + EPS
            iv = lax.bitcast_convert_type(var, jnp.int32)
            y = lax.bitcast_convert_type(jnp.int32(0x5F3759DF) - (iv >> 1),
                                         jnp.float32)
            y = y * (1.5 - 0.5 * var * y * y)
            y = y * (1.5 - 0.5 * var * y * y)
            y = y * (1.5 - 0.5 * var * y * y)
            for j in range(NJ):
                rows[i, pl.ds(L * j, L)] = (x[j] - mu) * (y * g[j]) + bt[j]
            return carry

        lax.fori_loop(0, SEQ, row_body, 0)

    def gather_start(rows, rbase, sem):
        pltpu.make_async_copy(
            table_h.at[idx_v.at[pl.ds(rbase, SPLIT)]],
            rows.at[pl.ds(0, SPLIT)], sem).start()
        pltpu.make_async_copy(
            table_h.at[idx_v.at[pl.ds(rbase + SPLIT, SEQ - SPLIT)]],
            rows.at[pl.ds(SPLIT, SEQ - SPLIT)], sem).start()

    def gather_wait(rows, sem):
        pltpu.make_async_copy(
            table_h.at[idx_v.at[pl.ds(0, SPLIT)]],
            rows.at[pl.ds(0, SPLIT)], sem).wait()
        pltpu.make_async_copy(
            table_h.at[idx_v.at[pl.ds(SPLIT, SEQ - SPLIT)]],
            rows.at[pl.ds(SPLIT, SEQ - SPLIT)], sem).wait()

    def write_start(rows, rbase, sem):
        pltpu.make_async_copy(
            rows, out_h.at[pl.ds(base + rbase, SEQ)], sem).start()

    def write_wait(rows, sem):
        pltpu.make_async_copy(
            rows, out_h.at[pl.ds(base, SEQ)], sem).wait()

    NPAIR = NBLK // 2
    gather_start(rows0, 0, gsem0)

    def pair_body(k, carry):
        b0 = 2 * k

        @pl.when(k > 0)
        def _():
            write_wait(rows1, osem1)

        gather_start(rows1, (b0 + 1) * SEQ, gsem1)
        gather_wait(rows0, gsem0)
        ln_rows(rows0)
        write_start(rows0, b0 * SEQ, osem0)
        gather_wait(rows1, gsem1)
        ln_rows(rows1)
        write_wait(rows0, osem0)

        @pl.when(k + 1 < NPAIR)
        def _():
            gather_start(rows0, (b0 + 2) * SEQ, gsem0)

        write_start(rows1, (b0 + 1) * SEQ, osem1)
        return carry

    lax.fori_loop(0, NPAIR, pair_body, 0)
    write_wait(rows1, osem1)


def kernel(text, text_table, pos_table, type_table, gamma, beta):
    out = _emb_ln_kernel(text.reshape(NTOK), text_table, pos_table,
                         type_table, gamma, beta)
    return out.reshape(BATCH, SEQ, HID)


# speedup vs baseline: 2.1363x; 1.7983x over previous
import functools

import jax
import jax.numpy as jnp
from jax import lax
from jax.experimental import pallas as pl
from jax.experimental.pallas import tpu as pltpu
from jax.experimental.pallas import tpu_sc as plsc

BATCH = 4096
SEQ = 200
HID = 128
EPS = 1e-12

NC = 2
NS = 16
NW = NC * NS
NTOK = BATCH * SEQ
RPW = NTOK // NW
NBLK = RPW // SEQ
L = 16
NJ = HID // L
SPLIT = 104

_mesh = plsc.VectorSubcoreMesh(core_axis_name="c", subcore_axis_name="s")


@functools.partial(
    pl.kernel,
    mesh=_mesh,
    out_type=jax.ShapeDtypeStruct((NTOK, HID), jnp.float32),
    scratch_types=[
        pltpu.VMEM((RPW,), jnp.int32),
        pltpu.VMEM((SEQ, HID), jnp.float32),
        pltpu.VMEM((SEQ, HID), jnp.float32),
        pltpu.VMEM((SEQ, HID), jnp.float32),
        pltpu.VMEM((2, HID), jnp.float32),
        pltpu.VMEM((HID,), jnp.float32),
        pltpu.VMEM((HID,), jnp.float32),
        pltpu.SemaphoreType.DMA,
        pltpu.SemaphoreType.DMA,
        pltpu.SemaphoreType.DMA,
        pltpu.SemaphoreType.DMA,
    ],
)
def _emb_ln_kernel(text_h, table_h, pos_h, type_h, gamma_h, beta_h, out_h,
                   idx_v, add_v, rows0, rows1, type_v, gam_v, bet_v,
                   gsem0, gsem1, osem0, osem1):
    wid = lax.axis_index("s") * NC + lax.axis_index("c")
    base = wid * RPW

    pltpu.sync_copy(text_h.at[pl.ds(base, RPW)], idx_v)
    pltpu.sync_copy(pos_h.at[pl.ds(0, SEQ)], add_v)
    pltpu.sync_copy(type_h, type_v)
    pltpu.sync_copy(gamma_h, gam_v)
    pltpu.sync_copy(beta_h, bet_v)

    t = [type_v[0, pl.ds(L * j, L)] for j in range(NJ)]

    def add_body(s, carry):
        for j in range(NJ):
            sl = pl.ds(L * j, L)
            add_v[s, sl] = add_v[s, sl] + t[j]
        return carry

    lax.fori_loop(0, SEQ, add_body, 0)

    g = [gam_v[pl.ds(L * j, L)] for j in range(NJ)]
    bt = [bet_v[pl.ds(L * j, L)] for j in range(NJ)]

    lane = lax.iota(jnp.int32, L)
    perms = [(lane ^ k)[:, None] for k in (8, 4, 2, 1)]
    gdn = lax.GatherDimensionNumbers(
        offset_dims=(), collapsed_slice_dims=(0,), start_index_map=(0,))

    def lane_sum(v):
        for p in perms:
            v = v + lax.gather(v, p, gdn, slice_sizes=(1,),
                               mode=lax.GatherScatterMode.PROMISE_IN_BOUNDS)
        return v

    def ln_rows(rows):
        @plsc.parallel_loop(0, SEQ, 1, unroll=4)
        def _row_body(i):
            x = [rows[i, pl.ds(L * j, L)] + add_v[i, pl.ds(L * j, L)]
                 for j in range(NJ)]
            s01 = (x[0] + x[1]) + (x[2] + x[3])
            s23 = (x[4] + x[5]) + (x[6] + x[7])
            tot = lane_sum(s01 + s23)
            q = [x[j] * x[j] for j in range(NJ)]
            q01 = (q[0] + q[1]) + (q[2] + q[3])
            q23 = (q[4] + q[5]) + (q[6] + q[7])
            ssq = lane_sum(q01 + q23)
            mu = tot * (1.0 / HID)
            var = ssq * (1.0 / HID) - mu * mu + EPS
            iv = lax.bitcast_convert_type(var, jnp.int32)
            y = lax.bitcast_convert_type(jnp.int32(0x5F3759DF) - (iv >> 1),
                                         jnp.float32)
            y = y * (1.5 - 0.5 * var * y * y)
            y = y * (1.5 - 0.5 * var * y * y)
            for j in range(NJ):
                rows[i, pl.ds(L * j, L)] = (x[j] - mu) * (y * g[j]) + bt[j]

    def gather_start(rows, rbase, sem):
        pltpu.make_async_copy(
            table_h.at[idx_v.at[pl.ds(rbase, SPLIT)]],
            rows.at[pl.ds(0, SPLIT)], sem).start()
        pltpu.make_async_copy(
            table_h.at[idx_v.at[pl.ds(rbase + SPLIT, SEQ - SPLIT)]],
            rows.at[pl.ds(SPLIT, SEQ - SPLIT)], sem).start()

    def gather_wait(rows, sem):
        pltpu.make_async_copy(
            table_h.at[idx_v.at[pl.ds(0, SPLIT)]],
            rows.at[pl.ds(0, SPLIT)], sem).wait()
        pltpu.make_async_copy(
            table_h.at[idx_v.at[pl.ds(SPLIT, SEQ - SPLIT)]],
            rows.at[pl.ds(SPLIT, SEQ - SPLIT)], sem).wait()

    def write_start(rows, rbase, sem):
        pltpu.make_async_copy(
            rows, out_h.at[pl.ds(base + rbase, SEQ)], sem).start()

    def write_wait(rows, sem):
        pltpu.make_async_copy(
            rows, out_h.at[pl.ds(base, SEQ)], sem).wait()

    NPAIR = NBLK // 2
    gather_start(rows0, 0, gsem0)

    def pair_body(k, carry):
        b0 = 2 * k

        @pl.when(k > 0)
        def _():
            write_wait(rows1, osem1)

        gather_start(rows1, (b0 + 1) * SEQ, gsem1)
        gather_wait(rows0, gsem0)
        ln_rows(rows0)
        write_start(rows0, b0 * SEQ, osem0)
        gather_wait(rows1, gsem1)
        ln_rows(rows1)
        write_wait(rows0, osem0)

        @pl.when(k + 1 < NPAIR)
        def _():
            gather_start(rows0, (b0 + 2) * SEQ, gsem0)

        write_start(rows1, (b0 + 1) * SEQ, osem1)
        return carry

    lax.fori_loop(0, NPAIR, pair_body, 0)
    write_wait(rows1, osem1)


def kernel(text, text_table, pos_table, type_table, gamma, beta):
    out = _emb_ln_kernel(text.reshape(NTOK), text_table, pos_table,
                         type_table, gamma, beta)
    return out.reshape(BATCH, SEQ, HID)
